# baseline (speedup 1.0000x reference)
import jax
import jax.numpy as jnp
from jax import lax
from jax.experimental import pallas as pl
from jax.experimental.pallas import tpu as pltpu

_B, _R, _C = 4, 4096, 2048
_RBLK = 512
_NR = _R // _RBLK
_NSLAB = _B * _NR


def _argmin_body(x_ref, o_ref, m_ref, i_ref):
    p = pl.program_id(0)
    r = lax.rem(p, _NR)
    for ch in range(2):
        cols = slice(ch * 1024, (ch + 1) * 1024)

        def scan_body(a, carry):
            amin, aidx = carry
            sl = x_ref[0, pl.ds(a * 8, 8), cols]
            took = sl < amin
            return jnp.minimum(amin, sl), jnp.where(took, a, aidx)

        init = (x_ref[0, 0:8, cols], jnp.zeros((8, 1024), jnp.int32))
        amin, aidx = jax.lax.fori_loop(1, _RBLK // 8, scan_body, init,
                                       unroll=4)

        rows = aidx * 8 + jax.lax.broadcasted_iota(jnp.int32, (8, 1024), 0)
        bm = jnp.min(amin, axis=0, keepdims=True)
        bidx = jnp.min(jnp.where(amin <= bm, rows, _R), axis=0,
                       keepdims=True) + r * _RBLK

        @pl.when(r == 0)
        def _init():
            m_ref[0:1, cols] = bm
            i_ref[0:1, cols] = bidx

        @pl.when(r > 0)
        def _merge():
            take = bm < m_ref[0:1, cols]
            i_ref[0:1, cols] = jnp.where(take, bidx, i_ref[0:1, cols])
            m_ref[0:1, cols] = jnp.where(take, bm, m_ref[0:1, cols])

        @pl.when(r == _NR - 1)
        def _emit():
            o_ref[0] = i_ref[...]


def kernel(x):
    xf = x.reshape(_NSLAB, _RBLK, _C)
    out = pl.pallas_call(
        _argmin_body,
        grid=(_NSLAB,),
        in_specs=[pl.BlockSpec((1, _RBLK, _C), lambda p: (p, 0, 0))],
        out_specs=pl.BlockSpec((1, 1, _C), lambda p: (p // _NR, 0, 0)),
        out_shape=jax.ShapeDtypeStruct((_B, 1, _C), jnp.int32),
        scratch_shapes=[
            pltpu.VMEM((1, _C), jnp.float32),
            pltpu.VMEM((1, _C), jnp.int32),
        ],
        compiler_params=pltpu.CompilerParams(
            dimension_semantics=("arbitrary",),
        ),
    )(xf)
    return out.reshape(_B, _C).astype(jnp.int64)


# speedup vs baseline: 1.1735x; 1.1735x over previous
import jax
import jax.numpy as jnp
from jax import lax
from jax.experimental import pallas as pl
from jax.experimental.pallas import tpu as pltpu

_B, _R, _C = 4, 4096, 2048
_RBLK = 1024
_NR = _R // _RBLK
_NSLAB = _B * _NR


def _argmin_body(x_ref, o_ref, m_ref, i_ref):
    p = pl.program_id(0)
    r = lax.rem(p, _NR)
    for ch in range(2):
        cols = slice(ch * 1024, (ch + 1) * 1024)

        def scan_body(a, carry):
            amin, aidx = carry
            sl = x_ref[0, pl.ds(a * 8, 8), cols]
            took = sl < amin
            return jnp.minimum(amin, sl), jnp.where(took, a, aidx)

        init = (x_ref[0, 0:8, cols], jnp.zeros((8, 1024), jnp.int32))
        amin, aidx = jax.lax.fori_loop(1, _RBLK // 8, scan_body, init,
                                       unroll=4)

        rows = aidx * 8 + jax.lax.broadcasted_iota(jnp.int32, (8, 1024), 0)
        bm = jnp.min(amin, axis=0, keepdims=True)
        bidx = jnp.min(jnp.where(amin <= bm, rows, _R), axis=0,
                       keepdims=True) + r * _RBLK

        @pl.when(r == 0)
        def _init():
            m_ref[0:1, cols] = bm
            i_ref[0:1, cols] = bidx

        @pl.when(r > 0)
        def _merge():
            take = bm < m_ref[0:1, cols]
            i_ref[0:1, cols] = jnp.where(take, bidx, i_ref[0:1, cols])
            m_ref[0:1, cols] = jnp.where(take, bm, m_ref[0:1, cols])

        @pl.when(r == _NR - 1)
        def _emit():
            o_ref[0] = i_ref[...]


def kernel(x):
    xf = x.reshape(_NSLAB, _RBLK, _C)
    out = pl.pallas_call(
        _argmin_body,
        grid=(_NSLAB,),
        in_specs=[pl.BlockSpec((1, _RBLK, _C), lambda p: (p, 0, 0))],
        out_specs=pl.BlockSpec((1, 1, _C), lambda p: (p // _NR, 0, 0)),
        out_shape=jax.ShapeDtypeStruct((_B, 1, _C), jnp.int32),
        scratch_shapes=[
            pltpu.VMEM((1, _C), jnp.float32),
            pltpu.VMEM((1, _C), jnp.int32),
        ],
        compiler_params=pltpu.CompilerParams(
            dimension_semantics=("arbitrary",),
        ),
    )(xf)
    return out.reshape(_B, _C).astype(jnp.int64)
